# baseline (device time: 64745 ns/iter reference)
import jax
import jax.numpy as jnp
from jax import lax
from jax.experimental import pallas as pl
from jax.experimental.pallas import tpu as pltpu

BLK = 256


def kernel(dy, W):
    m, f = dy.shape
    d = W.shape[0]
    my_y = lax.axis_index("y")
    my_z = lax.axis_index("z")
    q = my_y * 2 + my_z
    dy_blk = lax.dynamic_slice(dy, (q * BLK, 0), (BLK, f))

    def body(dy_ref, w_ref, out_ref, part_ref, xrecv_ref, send_sems, recv_sems):
        my_x = lax.axis_index("x")
        my_y = lax.axis_index("y")
        my_z = lax.axis_index("z")
        q = my_y * 2 + my_z

        barrier_sem = pltpu.get_barrier_semaphore()
        for nbr in (
            (1 - my_x, my_y, my_z),
            (my_x, my_y, 1 - my_z),
            (my_x, 1 - my_y, my_z),
        ):
            pl.semaphore_signal(
                barrier_sem, inc=1, device_id=nbr,
                device_id_type=pl.DeviceIdType.MESH,
            )
        pl.semaphore_wait(barrier_sem, 3)

        part_ref[...] = lax.dot_general(
            dy_ref[...], w_ref[...],
            dimension_numbers=(((1,), (1,)), ((), ())),
            preferred_element_type=jnp.float32,
        )

        rx = pltpu.make_async_remote_copy(
            src_ref=part_ref,
            dst_ref=xrecv_ref,
            send_sem=send_sems.at[0],
            recv_sem=recv_sems.at[0],
            device_id=(1 - my_x, my_y, my_z),
            device_id_type=pl.DeviceIdType.MESH,
        )
        rx.start()
        rx.wait()
        out_ref[pl.ds(q * BLK, BLK), :] = part_ref[...] + xrecv_ref[...]

        rz = pltpu.make_async_remote_copy(
            src_ref=out_ref.at[pl.ds(q * BLK, BLK)],
            dst_ref=out_ref.at[pl.ds(q * BLK, BLK)],
            send_sem=send_sems.at[1],
            recv_sem=recv_sems.at[1],
            device_id=(my_x, my_y, 1 - my_z),
            device_id_type=pl.DeviceIdType.MESH,
        )
        rz.start()
        rz.wait()

        ry = pltpu.make_async_remote_copy(
            src_ref=out_ref.at[pl.ds(my_y * (2 * BLK), 2 * BLK)],
            dst_ref=out_ref.at[pl.ds(my_y * (2 * BLK), 2 * BLK)],
            send_sem=send_sems.at[2],
            recv_sem=recv_sems.at[2],
            device_id=(my_x, 1 - my_y, my_z),
            device_id_type=pl.DeviceIdType.MESH,
        )
        ry.start()
        ry.wait()

    return pl.pallas_call(
        body,
        out_shape=jax.ShapeDtypeStruct((m, d), jnp.float32),
        in_specs=[
            pl.BlockSpec(memory_space=pltpu.VMEM),
            pl.BlockSpec(memory_space=pltpu.VMEM),
        ],
        out_specs=pl.BlockSpec(memory_space=pltpu.VMEM),
        scratch_shapes=[
            pltpu.VMEM((BLK, d), jnp.float32),
            pltpu.VMEM((BLK, d), jnp.float32),
            pltpu.SemaphoreType.DMA((3,)),
            pltpu.SemaphoreType.DMA((3,)),
        ],
        compiler_params=pltpu.CompilerParams(collective_id=0),
    )(dy_blk, W)


# device time: 42753 ns/iter; 1.5144x vs baseline; 1.5144x over previous
import jax
import jax.numpy as jnp
from jax import lax
from jax.experimental import pallas as pl
from jax.experimental.pallas import tpu as pltpu

BLK = 256
C = 4


def kernel(dy, W):
    m, f = dy.shape
    d = W.shape[0]
    cw = d // C
    my_y = lax.axis_index("y")
    my_z = lax.axis_index("z")
    q = my_y * 2 + my_z
    dy_blk = lax.dynamic_slice(dy, (q * BLK, 0), (BLK, f))

    def body(dy_ref, w_ref, out_ref, part_ref, xrecv_ref, send_sems, recv_sems):
        my_x = lax.axis_index("x")
        my_y = lax.axis_index("y")
        my_z = lax.axis_index("z")
        q = my_y * 2 + my_z
        qz = my_y * 2 + (1 - my_z)
        rowq = q * BLK
        rowqz = qz * BLK
        xdev = (1 - my_x, my_y, my_z)
        zdev = (my_x, my_y, 1 - my_z)
        ydev = (my_x, 1 - my_y, my_z)

        barrier_sem = pltpu.get_barrier_semaphore()
        for nbr in (xdev, zdev, ydev):
            pl.semaphore_signal(
                barrier_sem, inc=1, device_id=nbr,
                device_id_type=pl.DeviceIdType.MESH,
            )
        pl.semaphore_wait(barrier_sem, 3)

        def mk(src, dst, ph, c, dev):
            return pltpu.make_async_remote_copy(
                src_ref=src, dst_ref=dst,
                send_sem=send_sems.at[ph, c], recv_sem=recv_sems.at[ph, c],
                device_id=dev, device_id_type=pl.DeviceIdType.MESH,
            )

        x_rd, z_rd, y1_rd, y2_rd = [], [], [], []

        def compute_chunk(c):
            cs = pl.ds(c * cw, cw)
            part_ref[:, cs] = lax.dot_general(
                dy_ref[...], w_ref[cs, :],
                dimension_numbers=(((1,), (1,)), ((), ())),
                preferred_element_type=jnp.float32,
            )
            r = mk(part_ref.at[:, cs], xrecv_ref.at[:, cs], 0, c, xdev)
            r.start()
            x_rd.append(r)

        def reduce_chunk(c):
            cs = pl.ds(c * cw, cw)
            x_rd[c].wait()
            out_ref[pl.ds(rowq, BLK), cs] = part_ref[:, cs] + xrecv_ref[:, cs]
            rz = mk(out_ref.at[pl.ds(rowq, BLK), cs],
                    out_ref.at[pl.ds(rowq, BLK), cs], 1, c, zdev)
            rz.start()
            z_rd.append(rz)
            ry = mk(out_ref.at[pl.ds(rowq, BLK), cs],
                    out_ref.at[pl.ds(rowq, BLK), cs], 2, c, ydev)
            ry.start()
            y1_rd.append(ry)

        compute_chunk(0)
        for c in range(1, C):
            compute_chunk(c)
            reduce_chunk(c - 1)
        reduce_chunk(C - 1)

        for c in range(C):
            cs = pl.ds(c * cw, cw)
            z_rd[c].wait()
            ry2 = mk(out_ref.at[pl.ds(rowqz, BLK), cs],
                     out_ref.at[pl.ds(rowqz, BLK), cs], 3, c, ydev)
            ry2.start()
            y2_rd.append(ry2)

        for c in range(C):
            y1_rd[c].wait()
            y2_rd[c].wait()

    return pl.pallas_call(
        body,
        out_shape=jax.ShapeDtypeStruct((m, d), jnp.float32),
        in_specs=[
            pl.BlockSpec(memory_space=pltpu.VMEM),
            pl.BlockSpec(memory_space=pltpu.VMEM),
        ],
        out_specs=pl.BlockSpec(memory_space=pltpu.VMEM),
        scratch_shapes=[
            pltpu.VMEM((BLK, d), jnp.float32),
            pltpu.VMEM((BLK, d), jnp.float32),
            pltpu.SemaphoreType.DMA((4, C)),
            pltpu.SemaphoreType.DMA((4, C)),
        ],
        compiler_params=pltpu.CompilerParams(collective_id=0),
    )(dy_blk, W)


# device time: 37247 ns/iter; 1.7383x vs baseline; 1.1478x over previous
import jax
import jax.numpy as jnp
from jax import lax
from jax.experimental import pallas as pl
from jax.experimental.pallas import tpu as pltpu

BLK = 256
HBLK = 128
C = 4


def kernel(dy, W):
    m, f = dy.shape
    d = W.shape[0]
    cw = d // C
    my_y = lax.axis_index("y")
    my_z = lax.axis_index("z")
    q = my_y * 2 + my_z
    dy_blk = lax.dynamic_slice(dy, (q * BLK, 0), (BLK, f))

    def body(dy_ref, w_ref, out_ref, part_ref, xrecv_ref, send_sems, recv_sems):
        my_x = lax.axis_index("x")
        my_y = lax.axis_index("y")
        my_z = lax.axis_index("z")
        q = my_y * 2 + my_z
        qz = my_y * 2 + (1 - my_z)
        qy = (1 - my_y) * 2 + my_z
        rowq = q * BLK
        rowqz = qz * BLK
        rowqy = qy * BLK
        xdev = (1 - my_x, my_y, my_z)
        zdev = (my_x, my_y, 1 - my_z)
        ydev = (my_x, 1 - my_y, my_z)

        barrier_sem = pltpu.get_barrier_semaphore()
        for nbr in (xdev, zdev, ydev):
            pl.semaphore_signal(
                barrier_sem, inc=1, device_id=nbr,
                device_id_type=pl.DeviceIdType.MESH,
            )
        pl.semaphore_wait(barrier_sem, 3)

        PH_X, PH_Z1, PH_Y1, PH_Z2, PH_Y2 = range(5)

        def mk(src, dst, ph, c, dev):
            return pltpu.make_async_remote_copy(
                src_ref=src, dst_ref=dst,
                send_sem=send_sems.at[ph, c], recv_sem=recv_sems.at[ph, c],
                device_id=dev, device_id_type=pl.DeviceIdType.MESH,
            )

        x_rd, z1_rd, y1_rd, z2_rd, y2_rd = [], [], [], [], []

        def compute_chunk(c):
            cs = pl.ds(c * cw, cw)
            part_ref[:, cs] = lax.dot_general(
                dy_ref[...], w_ref[cs, :],
                dimension_numbers=(((1,), (1,)), ((), ())),
                preferred_element_type=jnp.float32,
            )
            r = mk(part_ref.at[:, cs], xrecv_ref.at[:, cs], PH_X, c, xdev)
            r.start()
            x_rd.append(r)

        def reduce_chunk(c):
            cs = pl.ds(c * cw, cw)
            x_rd[c].wait()
            out_ref[pl.ds(rowq, BLK), cs] = part_ref[:, cs] + xrecv_ref[:, cs]
            rz = mk(out_ref.at[pl.ds(rowq, BLK), cs],
                    out_ref.at[pl.ds(rowq, BLK), cs], PH_Z1, c, zdev)
            rz.start()
            z1_rd.append(rz)
            ry = mk(out_ref.at[pl.ds(rowq, BLK), cs],
                    out_ref.at[pl.ds(rowq, BLK), cs], PH_Y1, c, ydev)
            ry.start()
            y1_rd.append(ry)

        compute_chunk(0)
        for c in range(1, C):
            compute_chunk(c)
            reduce_chunk(c - 1)
        reduce_chunk(C - 1)

        for c in range(C):
            cs = pl.ds(c * cw, cw)
            z1_rd[c].wait()
            ry2 = mk(out_ref.at[pl.ds(rowqz + HBLK, HBLK), cs],
                     out_ref.at[pl.ds(rowqz + HBLK, HBLK), cs], PH_Y2, c, ydev)
            ry2.start()
            y2_rd.append(ry2)
            y1_rd[c].wait()
            rz2 = mk(out_ref.at[pl.ds(rowqy, HBLK), cs],
                     out_ref.at[pl.ds(rowqy, HBLK), cs], PH_Z2, c, zdev)
            rz2.start()
            z2_rd.append(rz2)

        for c in range(C):
            z2_rd[c].wait()
            y2_rd[c].wait()

    return pl.pallas_call(
        body,
        out_shape=jax.ShapeDtypeStruct((m, d), jnp.float32),
        in_specs=[
            pl.BlockSpec(memory_space=pltpu.VMEM),
            pl.BlockSpec(memory_space=pltpu.VMEM),
        ],
        out_specs=pl.BlockSpec(memory_space=pltpu.VMEM),
        scratch_shapes=[
            pltpu.VMEM((BLK, d), jnp.float32),
            pltpu.VMEM((BLK, d), jnp.float32),
            pltpu.SemaphoreType.DMA((5, C)),
            pltpu.SemaphoreType.DMA((5, C)),
        ],
        compiler_params=pltpu.CompilerParams(collective_id=0),
    )(dy_blk, W)


# device time: 35067 ns/iter; 1.8463x vs baseline; 1.0622x over previous
import jax
import jax.numpy as jnp
from jax import lax
from jax.experimental import pallas as pl
from jax.experimental.pallas import tpu as pltpu

BLK = 256
HBLK = 128
C = 8


def kernel(dy, W):
    m, f = dy.shape
    d = W.shape[0]
    cw = d // C
    my_y = lax.axis_index("y")
    my_z = lax.axis_index("z")
    q = my_y * 2 + my_z
    dy_blk = lax.dynamic_slice(dy, (q * BLK, 0), (BLK, f))

    def body(dy_ref, w_hbm, out_ref, part_ref, xrecv_ref, wbuf_ref,
             wcopy_sems, send_sems, recv_sems):
        my_x = lax.axis_index("x")
        my_y = lax.axis_index("y")
        my_z = lax.axis_index("z")
        q = my_y * 2 + my_z
        qz = my_y * 2 + (1 - my_z)
        qy = (1 - my_y) * 2 + my_z
        rowq = q * BLK
        rowqz = qz * BLK
        rowqy = qy * BLK
        xdev = (1 - my_x, my_y, my_z)
        zdev = (my_x, my_y, 1 - my_z)
        ydev = (my_x, 1 - my_y, my_z)

        def wcopy(c):
            return pltpu.make_async_copy(
                w_hbm.at[pl.ds(c * cw, cw), :],
                wbuf_ref.at[c % 2],
                wcopy_sems.at[c % 2],
            )

        wcopy(0).start()
        wcopy(1).start()

        barrier_sem = pltpu.get_barrier_semaphore()
        for nbr in (xdev, zdev, ydev):
            pl.semaphore_signal(
                barrier_sem, inc=1, device_id=nbr,
                device_id_type=pl.DeviceIdType.MESH,
            )
        pl.semaphore_wait(barrier_sem, 3)

        PH_X, PH_Z1, PH_Y1, PH_Z2, PH_Y2 = range(5)

        def mk(src, dst, ph, c, dev):
            return pltpu.make_async_remote_copy(
                src_ref=src, dst_ref=dst,
                send_sem=send_sems.at[ph, c], recv_sem=recv_sems.at[ph, c],
                device_id=dev, device_id_type=pl.DeviceIdType.MESH,
            )

        x_rd, z1_rd, y1_rd, z2_rd, y2_rd = [], [], [], [], []

        def compute_chunk(c):
            cs = pl.ds(c * cw, cw)
            wcopy(c).wait()
            part_ref[:, cs] = lax.dot_general(
                dy_ref[...], wbuf_ref[c % 2],
                dimension_numbers=(((1,), (1,)), ((), ())),
                preferred_element_type=jnp.float32,
            )
            r = mk(part_ref.at[:, cs], xrecv_ref.at[:, cs], PH_X, c, xdev)
            r.start()
            x_rd.append(r)
            if c + 2 < C:
                wcopy(c + 2).start()

        def reduce_chunk(c):
            cs = pl.ds(c * cw, cw)
            x_rd[c].wait()
            out_ref[pl.ds(rowq, BLK), cs] = part_ref[:, cs] + xrecv_ref[:, cs]
            rz = mk(out_ref.at[pl.ds(rowq, BLK), cs],
                    out_ref.at[pl.ds(rowq, BLK), cs], PH_Z1, c, zdev)
            rz.start()
            z1_rd.append(rz)
            ry = mk(out_ref.at[pl.ds(rowq, BLK), cs],
                    out_ref.at[pl.ds(rowq, BLK), cs], PH_Y1, c, ydev)
            ry.start()
            y1_rd.append(ry)

        compute_chunk(0)
        for c in range(1, C):
            compute_chunk(c)
            reduce_chunk(c - 1)
        reduce_chunk(C - 1)

        for c in range(C):
            cs = pl.ds(c * cw, cw)
            z1_rd[c].wait()
            ry2 = mk(out_ref.at[pl.ds(rowqz + HBLK, HBLK), cs],
                     out_ref.at[pl.ds(rowqz + HBLK, HBLK), cs], PH_Y2, c, ydev)
            ry2.start()
            y2_rd.append(ry2)
            y1_rd[c].wait()
            rz2 = mk(out_ref.at[pl.ds(rowqy, HBLK), cs],
                     out_ref.at[pl.ds(rowqy, HBLK), cs], PH_Z2, c, zdev)
            rz2.start()
            z2_rd.append(rz2)

        for c in range(C):
            z2_rd[c].wait()
            y2_rd[c].wait()

    return pl.pallas_call(
        body,
        out_shape=jax.ShapeDtypeStruct((m, d), jnp.float32),
        in_specs=[
            pl.BlockSpec(memory_space=pltpu.MemorySpace.VMEM),
            pl.BlockSpec(memory_space=pltpu.MemorySpace.HBM),
        ],
        out_specs=pl.BlockSpec(memory_space=pltpu.MemorySpace.VMEM),
        scratch_shapes=[
            pltpu.VMEM((BLK, d), jnp.float32),
            pltpu.VMEM((BLK, d), jnp.float32),
            pltpu.VMEM((2, d // C, f), jnp.float32),
            pltpu.SemaphoreType.DMA((2,)),
            pltpu.SemaphoreType.DMA((5, C)),
            pltpu.SemaphoreType.DMA((5, C)),
        ],
        compiler_params=pltpu.CompilerParams(collective_id=0),
    )(dy_blk, W)


# device time: 30878 ns/iter; 2.0968x vs baseline; 1.1357x over previous
import jax
import jax.numpy as jnp
from jax import lax
from jax.experimental import pallas as pl
from jax.experimental.pallas import tpu as pltpu

BLK = 256
HBLK = 128
C = 8
RED_LAG = 2
FWD_LAG = 4


def kernel(dy, W):
    m, f = dy.shape
    d = W.shape[0]
    cw = d // C
    my_y = lax.axis_index("y")
    my_z = lax.axis_index("z")
    q = my_y * 2 + my_z
    dy_blk = lax.dynamic_slice(dy, (q * BLK, 0), (BLK, f))

    def body(dy_ref, w_hbm, out_ref, part_ref, xrecv_ref, wbuf_ref,
             dy16_ref, w16_ref, wcopy_sems, send_sems, recv_sems):
        my_x = lax.axis_index("x")
        my_y = lax.axis_index("y")
        my_z = lax.axis_index("z")
        q = my_y * 2 + my_z
        qz = my_y * 2 + (1 - my_z)
        qy = (1 - my_y) * 2 + my_z
        rowq = q * BLK
        rowqz = qz * BLK
        rowqy = qy * BLK
        xdev = (1 - my_x, my_y, my_z)
        zdev = (my_x, my_y, 1 - my_z)
        ydev = (my_x, 1 - my_y, my_z)

        def wcopy(c):
            return pltpu.make_async_copy(
                w_hbm.at[pl.ds(c * cw, cw), :],
                wbuf_ref.at[c % 2],
                wcopy_sems.at[c % 2],
            )

        wcopy(0).start()
        wcopy(1).start()
        dy16_ref[...] = dy_ref[...].astype(jnp.bfloat16)

        barrier_sem = pltpu.get_barrier_semaphore()
        for nbr in (xdev, zdev, ydev):
            pl.semaphore_signal(
                barrier_sem, inc=1, device_id=nbr,
                device_id_type=pl.DeviceIdType.MESH,
            )
        pl.semaphore_wait(barrier_sem, 3)

        PH_X, PH_Z1, PH_Y1, PH_Z2, PH_Y2 = range(5)

        def mk(src, dst, ph, c, dev):
            return pltpu.make_async_remote_copy(
                src_ref=src, dst_ref=dst,
                send_sem=send_sems.at[ph, c], recv_sem=recv_sems.at[ph, c],
                device_id=dev, device_id_type=pl.DeviceIdType.MESH,
            )

        x_rd, z1_rd, y1_rd, z2_rd, y2_rd = [], [], [], [], []

        def compute_chunk(c):
            cs = pl.ds(c * cw, cw)
            wcopy(c).wait()
            w16_ref[...] = wbuf_ref[c % 2].astype(jnp.bfloat16)
            part_ref[:, cs] = lax.dot_general(
                dy16_ref[...], w16_ref[...],
                dimension_numbers=(((1,), (1,)), ((), ())),
                preferred_element_type=jnp.float32,
            )
            r = mk(part_ref.at[:, cs], xrecv_ref.at[:, cs], PH_X, c, xdev)
            r.start()
            x_rd.append(r)
            if c + 2 < C:
                wcopy(c + 2).start()

        def reduce_chunk(c):
            cs = pl.ds(c * cw, cw)
            x_rd[c].wait()
            out_ref[pl.ds(rowq, BLK), cs] = part_ref[:, cs] + xrecv_ref[:, cs]
            rz = mk(out_ref.at[pl.ds(rowq, BLK), cs],
                    out_ref.at[pl.ds(rowq, BLK), cs], PH_Z1, c, zdev)
            rz.start()
            z1_rd.append(rz)
            ry = mk(out_ref.at[pl.ds(rowq, BLK), cs],
                    out_ref.at[pl.ds(rowq, BLK), cs], PH_Y1, c, ydev)
            ry.start()
            y1_rd.append(ry)

        def forward_chunk(c):
            cs = pl.ds(c * cw, cw)
            z1_rd[c].wait()
            ry2 = mk(out_ref.at[pl.ds(rowqz + HBLK, HBLK), cs],
                     out_ref.at[pl.ds(rowqz + HBLK, HBLK), cs], PH_Y2, c, ydev)
            ry2.start()
            y2_rd.append(ry2)
            y1_rd[c].wait()
            rz2 = mk(out_ref.at[pl.ds(rowqy, HBLK), cs],
                     out_ref.at[pl.ds(rowqy, HBLK), cs], PH_Z2, c, zdev)
            rz2.start()
            z2_rd.append(rz2)

        for c in range(C):
            compute_chunk(c)
            if c >= RED_LAG:
                reduce_chunk(c - RED_LAG)
            if c >= FWD_LAG:
                forward_chunk(c - FWD_LAG)
        for c in range(C - RED_LAG, C):
            reduce_chunk(c)
        for c in range(C - FWD_LAG, C):
            forward_chunk(c)

        for c in range(C):
            z2_rd[c].wait()
            y2_rd[c].wait()

    return pl.pallas_call(
        body,
        out_shape=jax.ShapeDtypeStruct((m, d), jnp.float32),
        in_specs=[
            pl.BlockSpec(memory_space=pltpu.MemorySpace.VMEM),
            pl.BlockSpec(memory_space=pltpu.MemorySpace.HBM),
        ],
        out_specs=pl.BlockSpec(memory_space=pltpu.MemorySpace.VMEM),
        scratch_shapes=[
            pltpu.VMEM((BLK, d), jnp.float32),
            pltpu.VMEM((BLK, d), jnp.float32),
            pltpu.VMEM((2, d // C, f), jnp.float32),
            pltpu.VMEM((BLK, f), jnp.bfloat16),
            pltpu.VMEM((d // C, f), jnp.bfloat16),
            pltpu.SemaphoreType.DMA((2,)),
            pltpu.SemaphoreType.DMA((5, C)),
            pltpu.SemaphoreType.DMA((5, C)),
        ],
        compiler_params=pltpu.CompilerParams(collective_id=0),
    )(dy_blk, W)


# device time: 26424 ns/iter; 2.4502x vs baseline; 1.1686x over previous
import jax
import jax.numpy as jnp
from jax import lax
from jax.experimental import pallas as pl
from jax.experimental.pallas import tpu as pltpu

BLK = 256
HBLK = 128
C = 8
RED_LAG = 2
FWD_LAG = 4


def kernel(dy, W):
    m, f = dy.shape
    d = W.shape[0]
    cw = d // C
    my_y = lax.axis_index("y")
    my_z = lax.axis_index("z")
    q = my_y * 2 + my_z
    dy_blk = lax.dynamic_slice(dy, (q * BLK, 0), (BLK, f))

    def body(dy_ref, w_hbm, out_ref, part_ref, xrecv_ref, gat_ref, wbuf_ref,
             dy16_ref, w16_ref, wcopy_sems, send_sems, recv_sems):
        my_x = lax.axis_index("x")
        my_y = lax.axis_index("y")
        my_z = lax.axis_index("z")
        q = my_y * 2 + my_z
        qz = my_y * 2 + (1 - my_z)
        qy = (1 - my_y) * 2 + my_z
        qd = (1 - my_y) * 2 + (1 - my_z)
        rowq = q * BLK
        rowqz = qz * BLK
        rowqy = qy * BLK
        rowqd = qd * BLK
        xdev = (1 - my_x, my_y, my_z)
        zdev = (my_x, my_y, 1 - my_z)
        ydev = (my_x, 1 - my_y, my_z)

        def wcopy(c):
            return pltpu.make_async_copy(
                w_hbm.at[pl.ds(c * cw, cw), :],
                wbuf_ref.at[c % 2],
                wcopy_sems.at[c % 2],
            )

        wcopy(0).start()
        wcopy(1).start()
        dy16_ref[...] = dy_ref[...].astype(jnp.bfloat16)

        barrier_sem = pltpu.get_barrier_semaphore()
        for nbr in (xdev, zdev, ydev):
            pl.semaphore_signal(
                barrier_sem, inc=1, device_id=nbr,
                device_id_type=pl.DeviceIdType.MESH,
            )
        pl.semaphore_wait(barrier_sem, 3)

        PH_X, PH_Z1, PH_Y1, PH_Z2, PH_Y2 = range(5)

        def mk(src, dst, ph, c, dev):
            return pltpu.make_async_remote_copy(
                src_ref=src, dst_ref=dst,
                send_sem=send_sems.at[ph, c], recv_sem=recv_sems.at[ph, c],
                device_id=dev, device_id_type=pl.DeviceIdType.MESH,
            )

        x_rd, z1_rd, y1_rd, z2_rd, y2_rd = [], [], [], [], []

        def compute_chunk(c):
            cs = pl.ds(c * cw, cw)
            wcopy(c).wait()
            w16_ref[...] = wbuf_ref[c % 2].astype(jnp.bfloat16)
            part_ref[:, cs] = lax.dot_general(
                dy16_ref[...], w16_ref[...],
                dimension_numbers=(((1,), (1,)), ((), ())),
                preferred_element_type=jnp.float32,
            ).astype(jnp.bfloat16)
            r = mk(part_ref.at[:, cs], xrecv_ref.at[:, cs], PH_X, c, xdev)
            r.start()
            x_rd.append(r)
            if c + 2 < C:
                wcopy(c + 2).start()

        def reduce_chunk(c):
            cs = pl.ds(c * cw, cw)
            x_rd[c].wait()
            red = part_ref[:, cs] + xrecv_ref[:, cs]
            gat_ref[pl.ds(rowq, BLK), cs] = red
            out_ref[pl.ds(rowq, BLK), cs] = red.astype(jnp.float32)
            rz = mk(gat_ref.at[pl.ds(rowq, BLK), cs],
                    gat_ref.at[pl.ds(rowq, BLK), cs], PH_Z1, c, zdev)
            rz.start()
            z1_rd.append(rz)
            ry = mk(gat_ref.at[pl.ds(rowq, BLK), cs],
                    gat_ref.at[pl.ds(rowq, BLK), cs], PH_Y1, c, ydev)
            ry.start()
            y1_rd.append(ry)

        def forward_chunk(c):
            cs = pl.ds(c * cw, cw)
            z1_rd[c].wait()
            ry2 = mk(gat_ref.at[pl.ds(rowqz + HBLK, HBLK), cs],
                     gat_ref.at[pl.ds(rowqz + HBLK, HBLK), cs], PH_Y2, c, ydev)
            ry2.start()
            y2_rd.append(ry2)
            out_ref[pl.ds(rowqz, BLK), cs] = gat_ref[
                pl.ds(rowqz, BLK), cs].astype(jnp.float32)
            y1_rd[c].wait()
            rz2 = mk(gat_ref.at[pl.ds(rowqy, HBLK), cs],
                     gat_ref.at[pl.ds(rowqy, HBLK), cs], PH_Z2, c, zdev)
            rz2.start()
            z2_rd.append(rz2)
            out_ref[pl.ds(rowqy, BLK), cs] = gat_ref[
                pl.ds(rowqy, BLK), cs].astype(jnp.float32)

        for c in range(C):
            compute_chunk(c)
            if c >= RED_LAG:
                reduce_chunk(c - RED_LAG)
            if c >= FWD_LAG:
                forward_chunk(c - FWD_LAG)
        for c in range(C - RED_LAG, C):
            reduce_chunk(c)
        for c in range(C - FWD_LAG, C):
            forward_chunk(c)

        for c in range(C):
            cs = pl.ds(c * cw, cw)
            z2_rd[c].wait()
            y2_rd[c].wait()
            out_ref[pl.ds(rowqd, BLK), cs] = gat_ref[
                pl.ds(rowqd, BLK), cs].astype(jnp.float32)

    return pl.pallas_call(
        body,
        out_shape=jax.ShapeDtypeStruct((m, d), jnp.float32),
        in_specs=[
            pl.BlockSpec(memory_space=pltpu.MemorySpace.VMEM),
            pl.BlockSpec(memory_space=pltpu.MemorySpace.HBM),
        ],
        out_specs=pl.BlockSpec(memory_space=pltpu.MemorySpace.VMEM),
        scratch_shapes=[
            pltpu.VMEM((BLK, d), jnp.bfloat16),
            pltpu.VMEM((BLK, d), jnp.bfloat16),
            pltpu.VMEM((m, d), jnp.bfloat16),
            pltpu.VMEM((2, d // C, f), jnp.float32),
            pltpu.VMEM((BLK, f), jnp.bfloat16),
            pltpu.VMEM((d // C, f), jnp.bfloat16),
            pltpu.SemaphoreType.DMA((2,)),
            pltpu.SemaphoreType.DMA((5, C)),
            pltpu.SemaphoreType.DMA((5, C)),
        ],
        compiler_params=pltpu.CompilerParams(collective_id=0),
    )(dy_blk, W)


# device time: 24274 ns/iter; 2.6673x vs baseline; 1.0886x over previous
import jax
import jax.numpy as jnp
from jax import lax
from jax.experimental import pallas as pl
from jax.experimental.pallas import tpu as pltpu

BLK = 256
HBLK = 128
C = 8
P = C // 2
RED_LAG = 2
FWD_LAG = 4


def kernel(dy, W):
    m, f = dy.shape
    d = W.shape[0]
    cw = d // C
    pw = d // P
    my_y = lax.axis_index("y")
    my_z = lax.axis_index("z")
    q = my_y * 2 + my_z
    dy_blk = lax.dynamic_slice(dy, (q * BLK, 0), (BLK, f))

    def body(dy_ref, w_hbm, out_ref, part_ref, xrecv_ref, gat_ref, wbuf_ref,
             dy16_ref, w16_ref, wcopy_sems, send_sems, recv_sems):
        my_x = lax.axis_index("x")
        my_y = lax.axis_index("y")
        my_z = lax.axis_index("z")
        q = my_y * 2 + my_z
        qz = my_y * 2 + (1 - my_z)
        qy = (1 - my_y) * 2 + my_z
        rowq = q * BLK
        rowqz = qz * BLK
        rowqy = qy * BLK
        xdev = (1 - my_x, my_y, my_z)
        zdev = (my_x, my_y, 1 - my_z)
        ydev = (my_x, 1 - my_y, my_z)

        def wcopy(p):
            return pltpu.make_async_copy(
                w_hbm.at[pl.ds(p * pw, pw), :],
                wbuf_ref.at[p % 2],
                wcopy_sems.at[p % 2],
            )

        wcopy(0).start()
        wcopy(1).start()
        dy16_ref[...] = dy_ref[...].astype(jnp.bfloat16)

        barrier_sem = pltpu.get_barrier_semaphore()
        for nbr in (xdev, zdev, ydev):
            pl.semaphore_signal(
                barrier_sem, inc=1, device_id=nbr,
                device_id_type=pl.DeviceIdType.MESH,
            )
        pl.semaphore_wait(barrier_sem, 3)

        PH_X, PH_Z1, PH_Y1, PH_Z2, PH_Y2 = range(5)

        def mk(src, dst, ph, c, dev):
            return pltpu.make_async_remote_copy(
                src_ref=src, dst_ref=dst,
                send_sem=send_sems.at[ph, c], recv_sem=recv_sems.at[ph, c],
                device_id=dev, device_id_type=pl.DeviceIdType.MESH,
            )

        x_rd, z1_rd, y1_rd, z2_rd, y2_rd = [], [], [], [], []

        def compute_pair(p):
            ps = pl.ds(p * pw, pw)
            wcopy(p).wait()
            w16_ref[p % 2] = wbuf_ref[p % 2].astype(jnp.bfloat16)
            part_ref[:, ps] = lax.dot_general(
                dy16_ref[...], w16_ref[p % 2],
                dimension_numbers=(((1,), (1,)), ((), ())),
                preferred_element_type=jnp.float32,
            ).astype(jnp.bfloat16)
            for c in (2 * p, 2 * p + 1):
                cs = pl.ds(c * cw, cw)
                r = mk(part_ref.at[:, cs], xrecv_ref.at[:, cs], PH_X, c, xdev)
                r.start()
                x_rd.append(r)
            if p + 2 < P:
                wcopy(p + 2).start()

        def reduce_chunk(c):
            cs = pl.ds(c * cw, cw)
            x_rd[c].wait()
            gat_ref[pl.ds(rowq, BLK), cs] = part_ref[:, cs] + xrecv_ref[:, cs]
            rz = mk(gat_ref.at[pl.ds(rowq, BLK), cs],
                    gat_ref.at[pl.ds(rowq, BLK), cs], PH_Z1, c, zdev)
            rz.start()
            z1_rd.append(rz)
            ry = mk(gat_ref.at[pl.ds(rowq, BLK), cs],
                    gat_ref.at[pl.ds(rowq, BLK), cs], PH_Y1, c, ydev)
            ry.start()
            y1_rd.append(ry)

        def forward_chunk(c):
            cs = pl.ds(c * cw, cw)
            z1_rd[c].wait()
            ry2 = mk(gat_ref.at[pl.ds(rowqz + HBLK, HBLK), cs],
                     gat_ref.at[pl.ds(rowqz + HBLK, HBLK), cs], PH_Y2, c, ydev)
            ry2.start()
            y2_rd.append(ry2)
            y1_rd[c].wait()
            rz2 = mk(gat_ref.at[pl.ds(rowqy, HBLK), cs],
                     gat_ref.at[pl.ds(rowqy, HBLK), cs], PH_Z2, c, zdev)
            rz2.start()
            z2_rd.append(rz2)

        for p in range(P):
            compute_pair(p)
            done = 2 * p + 2
            while len(z1_rd) < done - RED_LAG:
                reduce_chunk(len(z1_rd))
            while len(y2_rd) < done - FWD_LAG:
                forward_chunk(len(y2_rd))
        while len(z1_rd) < C:
            reduce_chunk(len(z1_rd))
        while len(y2_rd) < C:
            forward_chunk(len(y2_rd))

        for c in range(C):
            z2_rd[c].wait()
            y2_rd[c].wait()

        out_ref[...] = gat_ref[...].astype(jnp.float32)

    return pl.pallas_call(
        body,
        out_shape=jax.ShapeDtypeStruct((m, d), jnp.float32),
        in_specs=[
            pl.BlockSpec(memory_space=pltpu.MemorySpace.VMEM),
            pl.BlockSpec(memory_space=pltpu.MemorySpace.HBM),
        ],
        out_specs=pl.BlockSpec(memory_space=pltpu.MemorySpace.VMEM),
        scratch_shapes=[
            pltpu.VMEM((BLK, d), jnp.bfloat16),
            pltpu.VMEM((BLK, d), jnp.bfloat16),
            pltpu.VMEM((m, d), jnp.bfloat16),
            pltpu.VMEM((2, d // P, f), jnp.float32),
            pltpu.VMEM((BLK, f), jnp.bfloat16),
            pltpu.VMEM((2, d // P, f), jnp.bfloat16),
            pltpu.SemaphoreType.DMA((2,)),
            pltpu.SemaphoreType.DMA((5, C)),
            pltpu.SemaphoreType.DMA((5, C)),
        ],
        compiler_params=pltpu.CompilerParams(collective_id=0),
    )(dy_blk, W)


# device time: 23905 ns/iter; 2.7084x vs baseline; 1.0154x over previous
import jax
import jax.numpy as jnp
from jax import lax
from jax.experimental import pallas as pl
from jax.experimental.pallas import tpu as pltpu

BLK = 256
HBLK = 128
C = 8
P = C // 2
RED_LAG = 2
FWD_LAG = 4


def kernel(dy, W):
    m, f = dy.shape
    d = W.shape[0]
    cw = d // C
    pw = d // P
    my_y = lax.axis_index("y")
    my_z = lax.axis_index("z")
    q = my_y * 2 + my_z
    dy_blk16 = lax.dynamic_slice(dy, (q * BLK, 0), (BLK, f)).astype(jnp.bfloat16)

    def body(dy16_ref, w_hbm, out_ref, part_ref, xrecv_ref, gat_ref, wbuf_ref,
             w16_ref, wcopy_sems, send_sems, recv_sems):
        my_x = lax.axis_index("x")
        my_y = lax.axis_index("y")
        my_z = lax.axis_index("z")
        q = my_y * 2 + my_z
        qz = my_y * 2 + (1 - my_z)
        qy = (1 - my_y) * 2 + my_z
        rowq = q * BLK
        rowqz = qz * BLK
        rowqy = qy * BLK
        xdev = (1 - my_x, my_y, my_z)
        zdev = (my_x, my_y, 1 - my_z)
        ydev = (my_x, 1 - my_y, my_z)

        def wcopy(p):
            return pltpu.make_async_copy(
                w_hbm.at[pl.ds(p * pw, pw), :],
                wbuf_ref.at[p % 2],
                wcopy_sems.at[p % 2],
            )

        wcopy(0).start()
        wcopy(1).start()

        barrier_sem = pltpu.get_barrier_semaphore()
        for nbr in (xdev, zdev, ydev):
            pl.semaphore_signal(
                barrier_sem, inc=1, device_id=nbr,
                device_id_type=pl.DeviceIdType.MESH,
            )
        pl.semaphore_wait(barrier_sem, 3)

        PH_X, PH_Z1, PH_Y1, PH_Z2, PH_Y2 = range(5)

        def mk(src, dst, ph, c, dev):
            return pltpu.make_async_remote_copy(
                src_ref=src, dst_ref=dst,
                send_sem=send_sems.at[ph, c], recv_sem=recv_sems.at[ph, c],
                device_id=dev, device_id_type=pl.DeviceIdType.MESH,
            )

        x_rd, z1_rd, y1_rd, z2_rd, y2_rd = [], [], [], [], []

        def compute_pair(p):
            ps = pl.ds(p * pw, pw)
            wcopy(p).wait()
            w16_ref[p % 2] = wbuf_ref[p % 2].astype(jnp.bfloat16)
            part_ref[:, ps] = lax.dot_general(
                dy16_ref[...], w16_ref[p % 2],
                dimension_numbers=(((1,), (1,)), ((), ())),
                preferred_element_type=jnp.float32,
            ).astype(jnp.bfloat16)
            for c in (2 * p, 2 * p + 1):
                cs = pl.ds(c * cw, cw)
                r = mk(part_ref.at[:, cs], xrecv_ref.at[:, cs], PH_X, c, xdev)
                r.start()
                x_rd.append(r)
            if p + 2 < P:
                wcopy(p + 2).start()

        def reduce_chunk(c):
            cs = pl.ds(c * cw, cw)
            x_rd[c].wait()
            gat_ref[pl.ds(rowq, BLK), cs] = part_ref[:, cs] + xrecv_ref[:, cs]
            rz = mk(gat_ref.at[pl.ds(rowq, BLK), cs],
                    gat_ref.at[pl.ds(rowq, BLK), cs], PH_Z1, c, zdev)
            rz.start()
            z1_rd.append(rz)
            ry = mk(gat_ref.at[pl.ds(rowq, BLK), cs],
                    gat_ref.at[pl.ds(rowq, BLK), cs], PH_Y1, c, ydev)
            ry.start()
            y1_rd.append(ry)

        def forward_chunk(c):
            cs = pl.ds(c * cw, cw)
            z1_rd[c].wait()
            ry2 = mk(gat_ref.at[pl.ds(rowqz + HBLK, HBLK), cs],
                     gat_ref.at[pl.ds(rowqz + HBLK, HBLK), cs], PH_Y2, c, ydev)
            ry2.start()
            y2_rd.append(ry2)
            y1_rd[c].wait()
            rz2 = mk(gat_ref.at[pl.ds(rowqy, HBLK), cs],
                     gat_ref.at[pl.ds(rowqy, HBLK), cs], PH_Z2, c, zdev)
            rz2.start()
            z2_rd.append(rz2)

        for p in range(P):
            compute_pair(p)
            done = 2 * p + 2
            while len(z1_rd) < done - RED_LAG:
                reduce_chunk(len(z1_rd))
            while len(y2_rd) < done - FWD_LAG:
                forward_chunk(len(y2_rd))
        while len(z1_rd) < C:
            reduce_chunk(len(z1_rd))
        while len(y2_rd) < C:
            forward_chunk(len(y2_rd))

        for row in (rowq, rowqz, rowqy):
            out_ref[pl.ds(row, BLK), :] = gat_ref[
                pl.ds(row, BLK), :].astype(jnp.float32)

        for c in range(C):
            z2_rd[c].wait()
            y2_rd[c].wait()
        rowqd = ((1 - my_y) * 2 + (1 - my_z)) * BLK
        out_ref[pl.ds(rowqd, BLK), :] = gat_ref[
            pl.ds(rowqd, BLK), :].astype(jnp.float32)

    return pl.pallas_call(
        body,
        out_shape=jax.ShapeDtypeStruct((m, d), jnp.float32),
        in_specs=[
            pl.BlockSpec(memory_space=pltpu.MemorySpace.VMEM),
            pl.BlockSpec(memory_space=pltpu.MemorySpace.HBM),
        ],
        out_specs=pl.BlockSpec(memory_space=pltpu.MemorySpace.VMEM),
        scratch_shapes=[
            pltpu.VMEM((BLK, d), jnp.bfloat16),
            pltpu.VMEM((BLK, d), jnp.bfloat16),
            pltpu.VMEM((m, d), jnp.bfloat16),
            pltpu.VMEM((2, d // P, f), jnp.float32),
            pltpu.VMEM((2, d // P, f), jnp.bfloat16),
            pltpu.SemaphoreType.DMA((2,)),
            pltpu.SemaphoreType.DMA((5, C)),
            pltpu.SemaphoreType.DMA((5, C)),
        ],
        compiler_params=pltpu.CompilerParams(collective_id=0),
    )(dy_blk16, W)
